# baseline (speedup 1.0000x reference)
import functools

import jax
import jax.numpy as jnp
from jax import lax
from jax.experimental import pallas as pl
from jax.experimental.pallas import tpu as pltpu
from jax.experimental.pallas import tpu_sc as plsc

N = 10000
E = 320000
NC = 2
NS = 16
NW = NC * NS
EPW = E // NW
K = 80
NCHUNK = EPW // K
NP = 10240
RPT = NP // NS
F = 128
ZR = 16
KP = 104
NCH = 98
EPT = NCH * KP


def _make_agg(P, with_count):
  mesh = plsc.VectorSubcoreMesh(core_axis_name="c", subcore_axis_name="s",
                                num_cores=NC, num_subcores=NS)
  out_type = [jax.ShapeDtypeStruct((NC, P, NP, F), jnp.float32)]
  if with_count:
    out_type.append(jax.ShapeDtypeStruct((NC, NP, F), jnp.float32))
  scratch = [
      pltpu.VMEM((1, KP), jnp.int32),
      pltpu.VMEM((1, KP), jnp.int32),
      pltpu.VMEM((1, KP), jnp.int32),
      pltpu.VMEM((1, KP), jnp.int32),
      pltpu.VMEM((KP, F), jnp.float32),
      pltpu.VMEM((KP, F), jnp.float32),
      pltpu.VMEM((ZR, F), jnp.float32),
      pltpu.VMEM_SHARED((NP, F), jnp.float32),
      pltpu.SemaphoreType.DMA,
      pltpu.SemaphoreType.DMA,
      pltpu.SemaphoreType.DMA,
      pltpu.SemaphoreType.DMA,
  ]

  def body(*refs):
    parts = refs[:P]
    srcr, dstr = refs[P], refs[P + 1]
    out = refs[P + 2]
    i = P + 3
    if with_count:
      outc = refs[i]
      i += 1
    s0, s1, d0, d1, r0, r1, zb, acc, sg0, sg1, ss0, ss1 = refs[i:i + 12]
    sbuf, dbuf, rbuf = (s0, s1), (d0, d1), (r0, r1)
    sg, ss = (sg0, sg1), (ss0, ss1)

    c = lax.axis_index("c")
    s = lax.axis_index("s")
    wid = c * NS + s

    zeros16 = jnp.zeros((16,), jnp.float32)
    ones16 = jnp.ones((16,), jnp.float32)

    def zrow(r, carry):
      for t in range(F // 16):
        zb[r, pl.ds(t * 16, 16)] = zeros16
      return carry

    lax.fori_loop(0, ZR, zrow, 0)

    def zstripe(q, carry):
      pltpu.sync_copy(zb, acc.at[pl.ds(s * RPT + q * ZR, ZR)])
      return carry

    for p in range(P):
      lax.fori_loop(0, RPT // ZR, zstripe, 0)
      plsc.subcore_barrier()

      for b in range(2):
        pltpu.sync_copy(srcr.at[wid, b], sbuf[b])
        pltpu.sync_copy(dstr.at[wid, b], dbuf[b])
        pltpu.async_copy(parts[p].at[sbuf[b].at[0]], rbuf[b], sg[b])

      def super_it(gg, carry):
        for b in range(2):
          g = 2 * gg + b
          pltpu.make_async_copy(parts[p].at[sbuf[b].at[0]],
                                rbuf[b], sg[b]).wait()
          pltpu.sync_copy(rbuf[b], acc.at[dbuf[b].at[0]], add=True)
          pltpu.sync_copy(srcr.at[wid, g + 2], sbuf[b])
          pltpu.sync_copy(dstr.at[wid, g + 2], dbuf[b])
          pltpu.async_copy(parts[p].at[sbuf[b].at[0]], rbuf[b], sg[b])
        return carry

      lax.fori_loop(0, (NCH - 2) // 2, super_it, 0)
      for b in range(2):
        pltpu.make_async_copy(parts[p].at[sbuf[b].at[0]],
                              rbuf[b], sg[b]).wait()
        pltpu.sync_copy(rbuf[b], acc.at[dbuf[b].at[0]], add=True)
      plsc.subcore_barrier()
      pltpu.sync_copy(acc.at[pl.ds(s * RPT, RPT)],
                      out.at[c, p, pl.ds(s * RPT, RPT)])

    if with_count:
      def orow(r, carry):
        for t in range(F // 16):
          r0[r, pl.ds(t * 16, 16)] = ones16
        return carry

      lax.fori_loop(0, KP, orow, 0)
      lax.fori_loop(0, RPT // ZR, zstripe, 0)
      plsc.subcore_barrier()

      for b in range(2):
        pltpu.sync_copy(dstr.at[wid, b], dbuf[b])
        pltpu.async_copy(r0, acc.at[dbuf[b].at[0]], ss[b], add=True)

      def csuper(gg, carry):
        for b in range(2):
          g = 2 * gg + b
          pltpu.make_async_copy(r0, acc.at[dbuf[b].at[0]], ss[b]).wait()
          pltpu.sync_copy(dstr.at[wid, g + 2], dbuf[b])
          pltpu.async_copy(r0, acc.at[dbuf[b].at[0]], ss[b], add=True)
        return carry

      lax.fori_loop(0, (NCH - 2) // 2, csuper, 0)
      for b in range(2):
        pltpu.make_async_copy(r0, acc.at[dbuf[b].at[0]], ss[b]).wait()
      plsc.subcore_barrier()
      pltpu.sync_copy(acc.at[pl.ds(s * RPT, RPT)],
                      outc.at[c, pl.ds(s * RPT, RPT)])

  return pl.kernel(body, out_type=out_type, mesh=mesh, scratch_types=scratch)


@functools.lru_cache(maxsize=None)
def _agg(P, with_count):
  return _make_agg(P, with_count)


def _tc_mean_body(P, s_ref, cnt_ref, out_ref):
  cnt = cnt_ref[0, :N, 0:1] + cnt_ref[1, :N, 0:1]
  inv = 1.0 / jnp.maximum(cnt, 1.0)
  parts = [s_ref[0, p, :N] + s_ref[1, p, :N] for p in range(P)]
  mean = jnp.concatenate(parts, axis=1) if P > 1 else parts[0]
  out_ref[...] = mean * inv


def _tc_sage_body(mean_ref, h_ref, wl, bl, wr, g, be, out_ref):
  z = (jnp.dot(mean_ref[...], wl[...], preferred_element_type=jnp.float32)
       + bl[...][None, :]
       + jnp.dot(h_ref[...], wr[...], preferred_element_type=jnp.float32))
  m = jnp.mean(z, axis=0)
  v = jnp.mean((z - m[None, :]) ** 2, axis=0)
  hn = (z - m[None, :]) * lax.rsqrt(v + 1e-5) * g[...][None, :] + be[...][None, :]
  out_ref[...] = jnp.maximum(hn, 0.0)


def _tc_head_body(h_ref, wc1, bc1, wc2, bc2, out_ref):
  c1 = jnp.maximum(
      jnp.dot(h_ref[...], wc1[...], preferred_element_type=jnp.float32)
      + bc1[...][None, :], 0.0)
  out_ref[...] = (jnp.dot(c1, wc2[...], preferred_element_type=jnp.float32)
                  + bc2[...][None, :])


def _tc_mean(P):
  return pl.pallas_call(
      functools.partial(_tc_mean_body, P),
      out_shape=jax.ShapeDtypeStruct((N, P * F), jnp.float32))


_tc_sage = pl.pallas_call(
    _tc_sage_body, out_shape=jax.ShapeDtypeStruct((N, 256), jnp.float32))
_tc_head = pl.pallas_call(
    _tc_head_body, out_shape=jax.ShapeDtypeStruct((N, 2), jnp.float32))


def kernel(x, edge_index, Wl0, bl0, Wr0, g0, be0, Wl1, bl1, Wr1, g1, be1,
           Wl2, bl2, Wr2, g2, be2, Wc1, bc1, Wc2, bc2):
  ei = edge_index.reshape(2, NW, EPW)
  pad = EPT - EPW
  src = jnp.concatenate(
      [ei[0], jnp.zeros((NW, pad), jnp.int32)], axis=1).reshape(
          NW, NCH, 1, KP)
  trash = (N + (jnp.arange(NW, dtype=jnp.int32) % NS))[:, None]
  dst = jnp.concatenate(
      [ei[1], jnp.broadcast_to(trash, (NW, pad))], axis=1).reshape(
          NW, NCH, 1, KP)

  s0, cnt = _agg(1, True)(x, src, dst)
  h1 = _tc_sage(_tc_mean(1)(s0, cnt), x, Wl0.T, bl0, Wr0.T, g0, be0)

  s1, = _agg(2, False)(h1[:, :F], h1[:, F:], src, dst)
  h2 = _tc_sage(_tc_mean(2)(s1, cnt), h1, Wl1.T, bl1, Wr1.T, g1, be1)

  s2, = _agg(2, False)(h2[:, :F], h2[:, F:], src, dst)
  h3 = _tc_sage(_tc_mean(2)(s2, cnt), h2, Wl2.T, bl2, Wr2.T, g2, be2)
  return _tc_head(h3, Wc1.T, bc1, Wc2.T, bc2)


# speedup vs baseline: 2.0588x; 2.0588x over previous
import functools

import jax
import jax.numpy as jnp
from jax import lax
from jax.experimental import pallas as pl
from jax.experimental.pallas import tpu as pltpu
from jax.experimental.pallas import tpu_sc as plsc

N = 10000
E = 320000
NC = 2
NS = 16
NW = NC * NS
EPW = E // NW
K = 80
NCHUNK = EPW // K
NP = 10240
RPT = NP // NS
F = 128
ZR = 16
KP = 100
NCH = 100
EPT = NCH * KP


def _make_agg(P, with_count):
  mesh = plsc.VectorSubcoreMesh(core_axis_name="c", subcore_axis_name="s",
                                num_cores=NC, num_subcores=NS)
  out_type = [jax.ShapeDtypeStruct((NC, P, NP, F), jnp.float32)]
  if with_count:
    out_type.append(jax.ShapeDtypeStruct((NC, NP, F), jnp.float32))
  scratch = [
      pltpu.VMEM((1, KP), jnp.int32),
      pltpu.VMEM((1, KP), jnp.int32),
      pltpu.VMEM((1, KP), jnp.int32),
      pltpu.VMEM((1, KP), jnp.int32),
      pltpu.VMEM((KP, F), jnp.float32),
      pltpu.VMEM((KP, F), jnp.float32),
      pltpu.VMEM((ZR, F), jnp.float32),
      pltpu.VMEM_SHARED((NP, F), jnp.float32),
      pltpu.SemaphoreType.DMA,
      pltpu.SemaphoreType.DMA,
      pltpu.SemaphoreType.DMA,
      pltpu.SemaphoreType.DMA,
  ]

  def body(*refs):
    parts = refs[:P]
    srcr, dstr = refs[P], refs[P + 1]
    out = refs[P + 2]
    i = P + 3
    if with_count:
      outc = refs[i]
      i += 1
    s0, s1, d0, d1, r0, r1, zb, acc, sg0, sg1, ss0, ss1 = refs[i:i + 12]
    sbuf, dbuf, rbuf = (s0, s1), (d0, d1), (r0, r1)
    sg, ss = (sg0, sg1), (ss0, ss1)

    c = lax.axis_index("c")
    s = lax.axis_index("s")
    wid = c * NS + s

    zeros16 = jnp.zeros((16,), jnp.float32)
    ones16 = jnp.ones((16,), jnp.float32)

    def zrow(r, carry):
      for t in range(F // 16):
        zb[r, pl.ds(t * 16, 16)] = zeros16
      return carry

    lax.fori_loop(0, ZR, zrow, 0)

    def zstripe(q, carry):
      pltpu.sync_copy(zb, acc.at[pl.ds(s * RPT + q * ZR, ZR)])
      return carry

    for p in range(P):
      lax.fori_loop(0, RPT // ZR, zstripe, 0)
      plsc.subcore_barrier()

      for b in range(2):
        pltpu.sync_copy(srcr.at[wid, b], sbuf[b])
        pltpu.sync_copy(dstr.at[wid, b], dbuf[b])
        pltpu.async_copy(parts[p].at[sbuf[b].at[0]], rbuf[b], sg[b])

      def super_it(gg, carry):
        for b in range(2):
          g = 2 * gg + b
          pltpu.make_async_copy(parts[p].at[sbuf[b].at[0]],
                                rbuf[b], sg[b]).wait()
          pltpu.sync_copy(rbuf[b], acc.at[dbuf[b].at[0]], add=True)
          pltpu.sync_copy(srcr.at[wid, g + 2], sbuf[b])
          pltpu.sync_copy(dstr.at[wid, g + 2], dbuf[b])
          pltpu.async_copy(parts[p].at[sbuf[b].at[0]], rbuf[b], sg[b])
        return carry

      lax.fori_loop(0, (NCH - 2) // 2, super_it, 0)
      for b in range(2):
        pltpu.make_async_copy(parts[p].at[sbuf[b].at[0]],
                              rbuf[b], sg[b]).wait()
        pltpu.sync_copy(rbuf[b], acc.at[dbuf[b].at[0]], add=True)
      plsc.subcore_barrier()
      pltpu.sync_copy(acc.at[pl.ds(s * RPT, RPT)],
                      out.at[c, p, pl.ds(s * RPT, RPT)])

    if with_count:
      def orow(r, carry):
        for t in range(F // 16):
          r0[r, pl.ds(t * 16, 16)] = ones16
        return carry

      lax.fori_loop(0, KP, orow, 0)
      lax.fori_loop(0, RPT // ZR, zstripe, 0)
      plsc.subcore_barrier()

      for b in range(2):
        pltpu.sync_copy(dstr.at[wid, b], dbuf[b])
        pltpu.async_copy(r0, acc.at[dbuf[b].at[0]], ss[b], add=True)

      def csuper(gg, carry):
        for b in range(2):
          g = 2 * gg + b
          pltpu.make_async_copy(r0, acc.at[dbuf[b].at[0]], ss[b]).wait()
          pltpu.sync_copy(dstr.at[wid, g + 2], dbuf[b])
          pltpu.async_copy(r0, acc.at[dbuf[b].at[0]], ss[b], add=True)
        return carry

      lax.fori_loop(0, (NCH - 2) // 2, csuper, 0)
      for b in range(2):
        pltpu.make_async_copy(r0, acc.at[dbuf[b].at[0]], ss[b]).wait()
      plsc.subcore_barrier()
      pltpu.sync_copy(acc.at[pl.ds(s * RPT, RPT)],
                      outc.at[c, pl.ds(s * RPT, RPT)])

  return pl.kernel(body, out_type=out_type, mesh=mesh, scratch_types=scratch)


@functools.lru_cache(maxsize=None)
def _agg(P, with_count):
  return _make_agg(P, with_count)


def _tc_mean_body(P, s_ref, cnt_ref, out_ref):
  cnt = cnt_ref[0, :N, 0:1] + cnt_ref[1, :N, 0:1]
  inv = 1.0 / jnp.maximum(cnt, 1.0)
  parts = [s_ref[0, p, :N] + s_ref[1, p, :N] for p in range(P)]
  mean = jnp.concatenate(parts, axis=1) if P > 1 else parts[0]
  out_ref[...] = mean * inv


def _tc_sage_body(mean_ref, h_ref, wl, bl, wr, g, be, out_ref):
  z = (jnp.dot(mean_ref[...], wl[...], preferred_element_type=jnp.float32)
       + bl[...][None, :]
       + jnp.dot(h_ref[...], wr[...], preferred_element_type=jnp.float32))
  m = jnp.mean(z, axis=0)
  v = jnp.mean((z - m[None, :]) ** 2, axis=0)
  hn = (z - m[None, :]) * lax.rsqrt(v + 1e-5) * g[...][None, :] + be[...][None, :]
  out_ref[...] = jnp.maximum(hn, 0.0)


def _tc_head_body(h_ref, wc1, bc1, wc2, bc2, out_ref):
  c1 = jnp.maximum(
      jnp.dot(h_ref[...], wc1[...], preferred_element_type=jnp.float32)
      + bc1[...][None, :], 0.0)
  out_ref[...] = (jnp.dot(c1, wc2[...], preferred_element_type=jnp.float32)
                  + bc2[...][None, :])


def _tc_mean(P):
  return pl.pallas_call(
      functools.partial(_tc_mean_body, P),
      out_shape=jax.ShapeDtypeStruct((N, P * F), jnp.float32))


_tc_sage = pl.pallas_call(
    _tc_sage_body, out_shape=jax.ShapeDtypeStruct((N, 256), jnp.float32))
_tc_head = pl.pallas_call(
    _tc_head_body, out_shape=jax.ShapeDtypeStruct((N, 2), jnp.float32))


def kernel(x, edge_index, Wl0, bl0, Wr0, g0, be0, Wl1, bl1, Wr1, g1, be1,
           Wl2, bl2, Wr2, g2, be2, Wc1, bc1, Wc2, bc2):
  ei = edge_index.reshape(2, NW, EPW)
  pad = EPT - EPW
  src = jnp.concatenate(
      [ei[0], jnp.zeros((NW, pad), jnp.int32)], axis=1).reshape(
          NW, NCH, 1, KP)
  trash = (N + (jnp.arange(NW, dtype=jnp.int32) % NS))[:, None]
  dst = jnp.concatenate(
      [ei[1], jnp.broadcast_to(trash, (NW, pad))], axis=1).reshape(
          NW, NCH, 1, KP)

  s0, cnt = _agg(1, True)(x, src, dst)
  h1 = _tc_sage(_tc_mean(1)(s0, cnt), x, Wl0.T, bl0, Wr0.T, g0, be0)

  s1, = _agg(2, False)(h1[:, :F], h1[:, F:], src, dst)
  h2 = _tc_sage(_tc_mean(2)(s1, cnt), h1, Wl1.T, bl1, Wr1.T, g1, be1)

  s2, = _agg(2, False)(h2[:, :F], h2[:, F:], src, dst)
  h3 = _tc_sage(_tc_mean(2)(s2, cnt), h2, Wl2.T, bl2, Wr2.T, g2, be2)
  return _tc_head(h3, Wc1.T, bc1, Wc2.T, bc2)


# speedup vs baseline: 2.4008x; 1.1661x over previous
import functools

import jax
import jax.numpy as jnp
from jax import lax
from jax.experimental import pallas as pl
from jax.experimental.pallas import tpu as pltpu
from jax.experimental.pallas import tpu_sc as plsc

N = 10000
E = 320000
NC = 2
NS = 16
NW = NC * NS
EPW = E // NW
K = 80
NCHUNK = EPW // K
NP = 10240
RPT = NP // NS
F = 128
ZR = 16
KP = 100
NCH = 100
EPT = NCH * KP


def _make_agg(P, with_count):
  mesh = plsc.VectorSubcoreMesh(core_axis_name="c", subcore_axis_name="s",
                                num_cores=NC, num_subcores=NS)
  out_type = [jax.ShapeDtypeStruct((NC, P, NP, F), jnp.float32)]
  if with_count:
    out_type.append(jax.ShapeDtypeStruct((NC, NP, F), jnp.float32))
  scratch = [
      pltpu.VMEM((2, 1, KP), jnp.int32),
      pltpu.VMEM((2, 1, KP), jnp.int32),
      pltpu.VMEM((KP, F), jnp.float32),
      pltpu.VMEM((KP, F), jnp.float32),
      pltpu.VMEM((ZR, F), jnp.float32),
      pltpu.VMEM_SHARED((NP, F), jnp.float32),
      pltpu.SemaphoreType.DMA,
      pltpu.SemaphoreType.DMA,
      pltpu.SemaphoreType.DMA,
      pltpu.SemaphoreType.DMA,
  ]

  def body(*refs):
    parts = refs[:P]
    edg = refs[P]
    out = refs[P + 1]
    i = P + 2
    if with_count:
      outc = refs[i]
      i += 1
    i0, i1, r0, r1, zb, acc, sg0, sg1, ss0, ss1 = refs[i:i + 10]
    ibuf, rbuf = (i0, i1), (r0, r1)
    sg, ss = (sg0, sg1), (ss0, ss1)

    c = lax.axis_index("c")
    s = lax.axis_index("s")
    wid = c * NS + s

    zeros16 = jnp.zeros((16,), jnp.float32)
    ones16 = jnp.ones((16,), jnp.float32)

    def zrow(r, carry):
      for t in range(F // 16):
        zb[r, pl.ds(t * 16, 16)] = zeros16
      return carry

    lax.fori_loop(0, ZR, zrow, 0)

    def zstripe(q, carry):
      pltpu.sync_copy(zb, acc.at[pl.ds(s * RPT + q * ZR, ZR)])
      return carry

    for p in range(P):
      lax.fori_loop(0, RPT // ZR, zstripe, 0)
      plsc.subcore_barrier()

      for b in range(2):
        pltpu.sync_copy(edg.at[wid, b], ibuf[b])
        pltpu.async_copy(parts[p].at[ibuf[b].at[0, 0]], rbuf[b], sg[b])

      def super_it(gg, carry):
        for b in range(2):
          g = 2 * gg + b
          pltpu.make_async_copy(parts[p].at[ibuf[b].at[0, 0]],
                                rbuf[b], sg[b]).wait()
          pltpu.sync_copy(rbuf[b], acc.at[ibuf[b].at[1, 0]], add=True)
          pltpu.sync_copy(edg.at[wid, g + 2], ibuf[b])
          pltpu.async_copy(parts[p].at[ibuf[b].at[0, 0]], rbuf[b], sg[b])
        return carry

      lax.fori_loop(0, (NCH - 2) // 2, super_it, 0)
      for b in range(2):
        pltpu.make_async_copy(parts[p].at[ibuf[b].at[0, 0]],
                              rbuf[b], sg[b]).wait()
        pltpu.sync_copy(rbuf[b], acc.at[ibuf[b].at[1, 0]], add=True)
      plsc.subcore_barrier()
      pltpu.sync_copy(acc.at[pl.ds(s * RPT, RPT)],
                      out.at[c, p, pl.ds(s * RPT, RPT)])

    if with_count:
      def orow(r, carry):
        for t in range(F // 16):
          r0[r, pl.ds(t * 16, 16)] = ones16
        return carry

      lax.fori_loop(0, KP, orow, 0)
      lax.fori_loop(0, RPT // ZR, zstripe, 0)
      plsc.subcore_barrier()

      for b in range(2):
        pltpu.sync_copy(edg.at[wid, b], ibuf[b])
        pltpu.async_copy(r0, acc.at[ibuf[b].at[1, 0]], ss[b], add=True)

      def csuper(gg, carry):
        for b in range(2):
          g = 2 * gg + b
          pltpu.make_async_copy(r0, acc.at[ibuf[b].at[1, 0]], ss[b]).wait()
          pltpu.sync_copy(edg.at[wid, g + 2], ibuf[b])
          pltpu.async_copy(r0, acc.at[ibuf[b].at[1, 0]], ss[b], add=True)
        return carry

      lax.fori_loop(0, (NCH - 2) // 2, csuper, 0)
      for b in range(2):
        pltpu.make_async_copy(r0, acc.at[ibuf[b].at[1, 0]], ss[b]).wait()
      plsc.subcore_barrier()
      pltpu.sync_copy(acc.at[pl.ds(s * RPT, RPT)],
                      outc.at[c, pl.ds(s * RPT, RPT)])

  return pl.kernel(body, out_type=out_type, mesh=mesh, scratch_types=scratch)


@functools.lru_cache(maxsize=None)
def _agg(P, with_count):
  return _make_agg(P, with_count)


def _tc_mean_body(P, s_ref, cnt_ref, out_ref):
  cnt = cnt_ref[0, :N, 0:1] + cnt_ref[1, :N, 0:1]
  inv = 1.0 / jnp.maximum(cnt, 1.0)
  parts = [s_ref[0, p, :N] + s_ref[1, p, :N] for p in range(P)]
  mean = jnp.concatenate(parts, axis=1) if P > 1 else parts[0]
  out_ref[...] = mean * inv


def _tc_sage_body(mean_ref, h_ref, wl, bl, wr, g, be, out_ref):
  z = (jnp.dot(mean_ref[...], wl[...], preferred_element_type=jnp.float32)
       + bl[...][None, :]
       + jnp.dot(h_ref[...], wr[...], preferred_element_type=jnp.float32))
  m = jnp.mean(z, axis=0)
  v = jnp.mean((z - m[None, :]) ** 2, axis=0)
  hn = (z - m[None, :]) * lax.rsqrt(v + 1e-5) * g[...][None, :] + be[...][None, :]
  out_ref[...] = jnp.maximum(hn, 0.0)


def _tc_head_body(h_ref, wc1, bc1, wc2, bc2, out_ref):
  c1 = jnp.maximum(
      jnp.dot(h_ref[...], wc1[...], preferred_element_type=jnp.float32)
      + bc1[...][None, :], 0.0)
  out_ref[...] = (jnp.dot(c1, wc2[...], preferred_element_type=jnp.float32)
                  + bc2[...][None, :])


def _tc_mean(P):
  return pl.pallas_call(
      functools.partial(_tc_mean_body, P),
      out_shape=jax.ShapeDtypeStruct((N, P * F), jnp.float32))


_tc_sage = pl.pallas_call(
    _tc_sage_body, out_shape=jax.ShapeDtypeStruct((N, 256), jnp.float32))
_tc_head = pl.pallas_call(
    _tc_head_body, out_shape=jax.ShapeDtypeStruct((N, 2), jnp.float32))


def kernel(x, edge_index, Wl0, bl0, Wr0, g0, be0, Wl1, bl1, Wr1, g1, be1,
           Wl2, bl2, Wr2, g2, be2, Wc1, bc1, Wc2, bc2):
  ei = edge_index.reshape(2, NW, EPW)
  pad = EPT - EPW
  if pad:
    trash = (N + (jnp.arange(NW, dtype=jnp.int32) % NS))[:, None]
    srcp = jnp.concatenate([ei[0], jnp.zeros((NW, pad), jnp.int32)], axis=1)
    dstp = jnp.concatenate(
        [ei[1], jnp.broadcast_to(trash, (NW, pad))], axis=1)
  else:
    srcp, dstp = ei[0], ei[1]
  edg = jnp.stack([srcp.reshape(NW, NCH, 1, KP),
                   dstp.reshape(NW, NCH, 1, KP)], axis=2)

  s0, cnt = _agg(1, True)(x, edg)
  h1 = _tc_sage(_tc_mean(1)(s0, cnt), x, Wl0.T, bl0, Wr0.T, g0, be0)

  s1, = _agg(2, False)(h1[:, :F], h1[:, F:], edg)
  h2 = _tc_sage(_tc_mean(2)(s1, cnt), h1, Wl1.T, bl1, Wr1.T, g1, be1)

  s2, = _agg(2, False)(h2[:, :F], h2[:, F:], edg)
  h3 = _tc_sage(_tc_mean(2)(s2, cnt), h2, Wl2.T, bl2, Wr2.T, g2, be2)
  return _tc_head(h3, Wc1.T, bc1, Wc2.T, bc2)


# speedup vs baseline: 2.4702x; 1.0289x over previous
import functools

import jax
import jax.numpy as jnp
from jax import lax
from jax.experimental import pallas as pl
from jax.experimental.pallas import tpu as pltpu
from jax.experimental.pallas import tpu_sc as plsc

N = 10000
E = 320000
NC = 2
NS = 16
NW = NC * NS
EPW = E // NW
K = 80
NCHUNK = EPW // K
NP = 10240
RPT = NP // NS
F = 128
ZR = 16
KP = 100
NCH = 100
EPT = NCH * KP


def _make_agg(P, with_count):
  mesh = plsc.VectorSubcoreMesh(core_axis_name="c", subcore_axis_name="s",
                                num_cores=NC, num_subcores=NS)
  out_type = [jax.ShapeDtypeStruct((NC, P, NP, F), jnp.float32)]
  if with_count:
    out_type.append(jax.ShapeDtypeStruct((NC, NP, F), jnp.float32))
  scratch = [
      pltpu.VMEM((2, 1, KP), jnp.int32),
      pltpu.VMEM((2, 1, KP), jnp.int32),
      pltpu.VMEM((KP, F), jnp.float32),
      pltpu.VMEM((KP, F), jnp.float32),
      pltpu.VMEM((ZR, F), jnp.float32),
      pltpu.VMEM_SHARED((NP, F), jnp.float32),
      pltpu.SemaphoreType.DMA,
      pltpu.SemaphoreType.DMA,
      pltpu.SemaphoreType.DMA,
      pltpu.SemaphoreType.DMA,
  ]

  def body(*refs):
    parts = refs[:P]
    edg = refs[P]
    out = refs[P + 1]
    i = P + 2
    if with_count:
      outc = refs[i]
      i += 1
    i0, i1, r0, r1, zb, acc, sg0, sg1, ss0, ss1 = refs[i:i + 10]
    ibuf, rbuf = (i0, i1), (r0, r1)
    sg, ss = (sg0, sg1), (ss0, ss1)

    c = lax.axis_index("c")
    s = lax.axis_index("s")
    wid = c * NS + s

    zeros16 = jnp.zeros((16,), jnp.float32)
    ones16 = jnp.ones((16,), jnp.float32)

    def zrow(r, carry):
      for t in range(F // 16):
        zb[r, pl.ds(t * 16, 16)] = zeros16
      return carry

    lax.fori_loop(0, ZR, zrow, 0)

    def zstripe(q, carry):
      pltpu.sync_copy(zb, acc.at[pl.ds(s * RPT + q * ZR, ZR)])
      return carry

    for p in range(P):
      lax.fori_loop(0, RPT // ZR, zstripe, 0)
      plsc.subcore_barrier()

      for b in range(2):
        pltpu.sync_copy(edg.at[wid, b], ibuf[b])
        pltpu.async_copy(parts[p].at[ibuf[b].at[0, 0]], rbuf[b], sg[b])

      def super_it(gg, carry):
        for b in range(2):
          g = 2 * gg + b
          pltpu.make_async_copy(parts[p].at[ibuf[b].at[0, 0]],
                                rbuf[b], sg[b]).wait()
          pltpu.sync_copy(rbuf[b], acc.at[ibuf[b].at[1, 0]], add=True)
          pltpu.sync_copy(edg.at[wid, g + 2], ibuf[b])
          pltpu.async_copy(parts[p].at[ibuf[b].at[0, 0]], rbuf[b], sg[b])
        return carry

      lax.fori_loop(0, (NCH - 2) // 2, super_it, 0)
      for b in range(2):
        pltpu.make_async_copy(parts[p].at[ibuf[b].at[0, 0]],
                              rbuf[b], sg[b]).wait()
        pltpu.sync_copy(rbuf[b], acc.at[ibuf[b].at[1, 0]], add=True)
      plsc.subcore_barrier()
      pltpu.sync_copy(acc.at[pl.ds(s * RPT, RPT)],
                      out.at[c, p, pl.ds(s * RPT, RPT)])

    if with_count:
      def orow(r, carry):
        for t in range(F // 16):
          r0[r, pl.ds(t * 16, 16)] = ones16
        return carry

      lax.fori_loop(0, KP, orow, 0)
      lax.fori_loop(0, RPT // ZR, zstripe, 0)
      plsc.subcore_barrier()

      for b in range(2):
        pltpu.sync_copy(edg.at[wid, b], ibuf[b])
        pltpu.async_copy(r0, acc.at[ibuf[b].at[1, 0]], ss[b], add=True)

      def csuper(gg, carry):
        for b in range(2):
          g = 2 * gg + b
          pltpu.make_async_copy(r0, acc.at[ibuf[b].at[1, 0]], ss[b]).wait()
          pltpu.sync_copy(edg.at[wid, g + 2], ibuf[b])
          pltpu.async_copy(r0, acc.at[ibuf[b].at[1, 0]], ss[b], add=True)
        return carry

      lax.fori_loop(0, (NCH - 2) // 2, csuper, 0)
      for b in range(2):
        pltpu.make_async_copy(r0, acc.at[ibuf[b].at[1, 0]], ss[b]).wait()
      plsc.subcore_barrier()
      pltpu.sync_copy(acc.at[pl.ds(s * RPT, RPT)],
                      outc.at[c, pl.ds(s * RPT, RPT)])

  return pl.kernel(body, out_type=out_type, mesh=mesh, scratch_types=scratch)


@functools.lru_cache(maxsize=None)
def _agg(P, with_count):
  return _make_agg(P, with_count)


def _tc_inv_body(cnt_ref, out_ref):
  cnt = cnt_ref[0, :N, 0:1] + cnt_ref[1, :N, 0:1]
  out_ref[...] = 1.0 / jnp.maximum(cnt, 1.0)


def _tc_sage_body(P, s_ref, inv_ref, h_ref, wl, bl, wr, g, be, out_ref):
  parts = [s_ref[0, p, :N] + s_ref[1, p, :N] for p in range(P)]
  mean = (jnp.concatenate(parts, axis=1) if P > 1 else parts[0]) * inv_ref[...]
  z = (jnp.dot(mean, wl[...], preferred_element_type=jnp.float32)
       + bl[...][None, :]
       + jnp.dot(h_ref[...], wr[...], preferred_element_type=jnp.float32))
  m = jnp.mean(z, axis=0)
  v = jnp.mean((z - m[None, :]) ** 2, axis=0)
  hn = (z - m[None, :]) * lax.rsqrt(v + 1e-5) * g[...][None, :] + be[...][None, :]
  out_ref[...] = jnp.maximum(hn, 0.0)


def _tc_head_body(h_ref, wc1, bc1, wc2, bc2, out_ref):
  c1 = jnp.maximum(
      jnp.dot(h_ref[...], wc1[...], preferred_element_type=jnp.float32)
      + bc1[...][None, :], 0.0)
  out_ref[...] = (jnp.dot(c1, wc2[...], preferred_element_type=jnp.float32)
                  + bc2[...][None, :])


_tc_inv = pl.pallas_call(
    _tc_inv_body, out_shape=jax.ShapeDtypeStruct((N, 1), jnp.float32))


def _tc_sage(P, D):
  return pl.pallas_call(
      functools.partial(_tc_sage_body, P),
      grid=(2,),
      in_specs=[
          pl.BlockSpec((NC, P, NP, F), lambda j: (0, 0, 0, 0)),
          pl.BlockSpec((N, 1), lambda j: (0, 0)),
          pl.BlockSpec((N, D), lambda j: (0, 0)),
          pl.BlockSpec((D, F), lambda j: (0, j)),
          pl.BlockSpec((F,), lambda j: (j,)),
          pl.BlockSpec((D, F), lambda j: (0, j)),
          pl.BlockSpec((F,), lambda j: (j,)),
          pl.BlockSpec((F,), lambda j: (j,)),
      ],
      out_specs=pl.BlockSpec((N, F), lambda j: (0, j)),
      out_shape=jax.ShapeDtypeStruct((N, 256), jnp.float32))
_tc_head = pl.pallas_call(
    _tc_head_body, out_shape=jax.ShapeDtypeStruct((N, 2), jnp.float32))


def kernel(x, edge_index, Wl0, bl0, Wr0, g0, be0, Wl1, bl1, Wr1, g1, be1,
           Wl2, bl2, Wr2, g2, be2, Wc1, bc1, Wc2, bc2):
  ei = edge_index.reshape(2, NW, EPW)
  pad = EPT - EPW
  if pad:
    trash = (N + (jnp.arange(NW, dtype=jnp.int32) % NS))[:, None]
    srcp = jnp.concatenate([ei[0], jnp.zeros((NW, pad), jnp.int32)], axis=1)
    dstp = jnp.concatenate(
        [ei[1], jnp.broadcast_to(trash, (NW, pad))], axis=1)
  else:
    srcp, dstp = ei[0], ei[1]
  edg = jnp.stack([srcp.reshape(NW, NCH, 1, KP),
                   dstp.reshape(NW, NCH, 1, KP)], axis=2)

  s0, cnt = _agg(1, True)(x, edg)
  inv = _tc_inv(cnt)
  h1 = _tc_sage(1, 128)(s0, inv, x, Wl0.T, bl0, Wr0.T, g0, be0)

  s1, = _agg(2, False)(h1[:, :F], h1[:, F:], edg)
  h2 = _tc_sage(2, 256)(s1, inv, h1, Wl1.T, bl1, Wr1.T, g1, be1)

  s2, = _agg(2, False)(h2[:, :F], h2[:, F:], edg)
  h3 = _tc_sage(2, 256)(s2, inv, h2, Wl2.T, bl2, Wr2.T, g2, be2)
  return _tc_head(h3, Wc1.T, bc1, Wc2.T, bc2)


# speedup vs baseline: 2.4726x; 1.0010x over previous
import functools

import jax
import jax.numpy as jnp
from jax import lax
from jax.experimental import pallas as pl
from jax.experimental.pallas import tpu as pltpu
from jax.experimental.pallas import tpu_sc as plsc

N = 10000
E = 320000
NC = 2
NS = 16
NW = NC * NS
EPW = E // NW
NP = 10240
RPT = NP // NS
F = 128
ZR = 16
KP = 100
NCH = 100
EPT = NCH * KP


def _make_agg(P, with_count):
  mesh = plsc.VectorSubcoreMesh(core_axis_name="c", subcore_axis_name="s",
                                num_cores=NC, num_subcores=NS)
  out_type = [jax.ShapeDtypeStruct((NC, P, NP, F), jnp.float32)]
  if with_count:
    out_type.append(jax.ShapeDtypeStruct((NC, NP, F), jnp.float32))
  scratch = [
      pltpu.VMEM((2, 1, KP), jnp.int32),
      pltpu.VMEM((2, 1, KP), jnp.int32),
      pltpu.VMEM((KP, F), jnp.float32),
      pltpu.VMEM((KP, F), jnp.float32),
      pltpu.VMEM((ZR, F), jnp.float32),
      pltpu.VMEM_SHARED((NP, F), jnp.float32),
      pltpu.SemaphoreType.DMA,
      pltpu.SemaphoreType.DMA,
      pltpu.SemaphoreType.DMA,
      pltpu.SemaphoreType.DMA,
  ]

  def body(*refs):
    parts = refs[:P]
    edg = refs[P]
    out = refs[P + 1]
    i = P + 2
    if with_count:
      outc = refs[i]
      i += 1
    i0, i1, r0, r1, zb, acc, sg0, sg1, ss0, ss1 = refs[i:i + 10]
    ibuf, rbuf = (i0, i1), (r0, r1)
    sg, ss = (sg0, sg1), (ss0, ss1)

    c = lax.axis_index("c")
    s = lax.axis_index("s")
    wid = c * NS + s

    zeros16 = jnp.zeros((16,), jnp.float32)
    ones16 = jnp.ones((16,), jnp.float32)

    def zrow(r, carry):
      for t in range(F // 16):
        zb[r, pl.ds(t * 16, 16)] = zeros16
      return carry

    lax.fori_loop(0, ZR, zrow, 0)

    def zstripe(q, carry):
      pltpu.sync_copy(zb, acc.at[pl.ds(s * RPT + q * ZR, ZR)])
      return carry

    for p in range(P):
      lax.fori_loop(0, RPT // ZR, zstripe, 0)
      plsc.subcore_barrier()

      for b in range(2):
        pltpu.sync_copy(edg.at[wid, b], ibuf[b])
        pltpu.async_copy(parts[p].at[ibuf[b].at[0, 0]], rbuf[b], sg[b])

      def super_it(gg, carry):
        for b in range(2):
          g = 2 * gg + b
          pltpu.make_async_copy(parts[p].at[ibuf[b].at[0, 0]],
                                rbuf[b], sg[b]).wait()
          pltpu.sync_copy(rbuf[b], acc.at[ibuf[b].at[1, 0]], add=True)
          pltpu.sync_copy(edg.at[wid, g + 2], ibuf[b])
          pltpu.async_copy(parts[p].at[ibuf[b].at[0, 0]], rbuf[b], sg[b])
        return carry

      lax.fori_loop(0, (NCH - 2) // 2, super_it, 0)
      for b in range(2):
        pltpu.make_async_copy(parts[p].at[ibuf[b].at[0, 0]],
                              rbuf[b], sg[b]).wait()
        pltpu.sync_copy(rbuf[b], acc.at[ibuf[b].at[1, 0]], add=True)
      plsc.subcore_barrier()
      pltpu.sync_copy(acc.at[pl.ds(s * RPT, RPT)],
                      out.at[c, p, pl.ds(s * RPT, RPT)])

    if with_count:
      def orow(r, carry):
        for t in range(F // 16):
          r0[r, pl.ds(t * 16, 16)] = ones16
        return carry

      lax.fori_loop(0, KP, orow, 0)
      lax.fori_loop(0, RPT // ZR, zstripe, 0)
      plsc.subcore_barrier()

      for b in range(2):
        pltpu.sync_copy(edg.at[wid, b], ibuf[b])
        pltpu.async_copy(r0, acc.at[ibuf[b].at[1, 0]], ss[b], add=True)

      def csuper(gg, carry):
        for b in range(2):
          g = 2 * gg + b
          pltpu.make_async_copy(r0, acc.at[ibuf[b].at[1, 0]], ss[b]).wait()
          pltpu.sync_copy(edg.at[wid, g + 2], ibuf[b])
          pltpu.async_copy(r0, acc.at[ibuf[b].at[1, 0]], ss[b], add=True)
        return carry

      lax.fori_loop(0, (NCH - 2) // 2, csuper, 0)
      for b in range(2):
        pltpu.make_async_copy(r0, acc.at[ibuf[b].at[1, 0]], ss[b]).wait()
      plsc.subcore_barrier()
      pltpu.sync_copy(acc.at[pl.ds(s * RPT, RPT)],
                      outc.at[c, pl.ds(s * RPT, RPT)])

  return pl.kernel(body, out_type=out_type, mesh=mesh, scratch_types=scratch)


@functools.lru_cache(maxsize=None)
def _agg(P, with_count):
  return _make_agg(P, with_count)


def _tc_inv_body(cnt_ref, out_ref):
  cnt = cnt_ref[0, :N, 0:1] + cnt_ref[1, :N, 0:1]
  out_ref[...] = 1.0 / jnp.maximum(cnt, 1.0)


def _tc_sage_body(P, s_ref, inv_ref, h_ref, wl, bl, wr, g, be, out_ref):
  parts = [s_ref[0, p, :N] + s_ref[1, p, :N] for p in range(P)]
  mean = (jnp.concatenate(parts, axis=1) if P > 1 else parts[0]) * inv_ref[...]
  z = (jnp.dot(mean, wl[...], preferred_element_type=jnp.float32)
       + bl[...][None, :]
       + jnp.dot(h_ref[...], wr[...], preferred_element_type=jnp.float32))
  m = jnp.mean(z, axis=0)
  v = jnp.mean((z - m[None, :]) ** 2, axis=0)
  hn = (z - m[None, :]) * lax.rsqrt(v + 1e-5) * g[...][None, :] + be[...][None, :]
  out_ref[...] = jnp.maximum(hn, 0.0)


def _tc_head_body(h_ref, wc1, bc1, wc2, bc2, out_ref):
  c1 = jnp.maximum(
      jnp.dot(h_ref[...], wc1[...], preferred_element_type=jnp.float32)
      + bc1[...][None, :], 0.0)
  out_ref[...] = (jnp.dot(c1, wc2[...], preferred_element_type=jnp.float32)
                  + bc2[...][None, :])


_tc_inv = pl.pallas_call(
    _tc_inv_body, out_shape=jax.ShapeDtypeStruct((N, 1), jnp.float32))


def _tc_sage(P, D):
  return pl.pallas_call(
      functools.partial(_tc_sage_body, P),
      grid=(2,),
      in_specs=[
          pl.BlockSpec((NC, P, NP, F), lambda j: (0, 0, 0, 0)),
          pl.BlockSpec((N, 1), lambda j: (0, 0)),
          pl.BlockSpec((N, D), lambda j: (0, 0)),
          pl.BlockSpec((D, F), lambda j: (0, j)),
          pl.BlockSpec((F,), lambda j: (j,)),
          pl.BlockSpec((D, F), lambda j: (0, j)),
          pl.BlockSpec((F,), lambda j: (j,)),
          pl.BlockSpec((F,), lambda j: (j,)),
      ],
      out_specs=pl.BlockSpec((N, F), lambda j: (0, j)),
      out_shape=jax.ShapeDtypeStruct((N, 256), jnp.float32))
_tc_head = pl.pallas_call(
    _tc_head_body, out_shape=jax.ShapeDtypeStruct((N, 2), jnp.float32))


def kernel(x, edge_index, Wl0, bl0, Wr0, g0, be0, Wl1, bl1, Wr1, g1, be1,
           Wl2, bl2, Wr2, g2, be2, Wc1, bc1, Wc2, bc2):
  ei = edge_index.reshape(2, NW, EPW)
  pad = EPT - EPW
  if pad:
    trash = (N + (jnp.arange(NW, dtype=jnp.int32) % NS))[:, None]
    srcp = jnp.concatenate([ei[0], jnp.zeros((NW, pad), jnp.int32)], axis=1)
    dstp = jnp.concatenate(
        [ei[1], jnp.broadcast_to(trash, (NW, pad))], axis=1)
  else:
    srcp, dstp = ei[0], ei[1]
  edg = jnp.stack([srcp.reshape(NW, NCH, 1, KP),
                   dstp.reshape(NW, NCH, 1, KP)], axis=2)

  s0, cnt = _agg(1, True)(x, edg)
  inv = _tc_inv(cnt)
  h1 = _tc_sage(1, 128)(s0, inv, x, Wl0.T, bl0, Wr0.T, g0, be0)

  s1, = _agg(2, False)(h1[:, :F], h1[:, F:], edg)
  h2 = _tc_sage(2, 256)(s1, inv, h1, Wl1.T, bl1, Wr1.T, g1, be1)

  s2, = _agg(2, False)(h2[:, :F], h2[:, F:], edg)
  h3 = _tc_sage(2, 256)(s2, inv, h2, Wl2.T, bl2, Wr2.T, g2, be2)
  return _tc_head(h3, Wc1.T, bc1, Wc2.T, bc2)
